# baseline (speedup 1.0000x reference)
import jax
import jax.numpy as jnp
from jax.experimental import pallas as pl
from jax.experimental.pallas import tpu as pltpu

_ML = 12
_B = 16
_S = 768
_T = (_ML + 1) * _ML // 2
_ROWS = _T * _B


def _off(L):
    return _T - (_ML - L + 2) * (_ML - L + 1) // 2


def _r(L):
    return _off(L) * _B


def _body(we, wfull, b1, w2t, b2, rb,
          ivec, isc, ovec, osc,
          ipj, opj):
    f32 = jnp.float32
    bf16 = jnp.bfloat16
    wfull_v = wfull[...].astype(bf16)
    b1_v = b1[...]
    w2t_v = w2t[...].astype(bf16)
    b2_v = b2[...]

    def level_update(n_slots, n16, lpa, rpr, sl, sr, vec_ref, sc_ref, row0):
        h = jax.nn.relu(lpa[:, :_S].astype(f32) + rpr[:, :_S].astype(f32) + b1_v)
        v = jax.nn.relu(jnp.dot(h.astype(bf16), w2t_v,
                                preferred_element_type=f32) + b2_v)
        s = jnp.sum(lpa[:, _S:].astype(f32) * rpr[:, _S:].astype(f32),
                    axis=1, keepdims=True) + sl + sr
        s3 = s.reshape(n_slots, n16, 1)
        m = jnp.max(s3, axis=0, keepdims=True)
        e = jnp.exp(s3 - m)
        w3 = e / jnp.sum(e, axis=0, keepdims=True)
        out_s = jnp.sum(w3 * s3, axis=0)
        out_v = jnp.sum(w3 * v.reshape(n_slots, n16, _S), axis=0)
        vec_ref[row0:row0 + n16, :] = out_v
        sc_ref[row0:row0 + n16, :] = out_s
        return out_v

    we_v = we[...]
    ivec[0:_ML * _B, :] = we_v
    isc[0:_ML * _B, :] = jnp.zeros((_ML * _B, 1), f32)
    ipj[0:_ML * _B, 0:3 * _S] = jnp.dot(we_v.astype(bf16), wfull_v,
                                        preferred_element_type=f32).astype(bf16)
    ipj[0:_ML * _B, 3 * _S:] = we_v.astype(bf16)

    for L in range(2, _ML + 1):
        nc = _ML + 1 - L
        n16 = nc * _B
        c = L - 1
        lml, rl, sll, srl = [], [], [], []
        for ci in range(c):
            a = _r(1 + ci)
            b = _r(L - 1 - ci) + (1 + ci) * _B
            lml.append(ipj[a:a + n16, 0:2 * _S])
            rl.append(ipj[b:b + n16, 2 * _S:])
            sll.append(isc[a:a + n16, :])
            srl.append(isc[b:b + n16, :])
        out_v = level_update(
            c, n16,
            jnp.concatenate(lml, axis=0), jnp.concatenate(rl, axis=0),
            jnp.concatenate(sll, axis=0), jnp.concatenate(srl, axis=0),
            ivec, isc, _r(L))
        row0 = _r(L)
        ipj[row0:row0 + n16, 0:3 * _S] = jnp.dot(
            out_v.astype(bf16), wfull_v, preferred_element_type=f32).astype(bf16)
        ipj[row0:row0 + n16, 3 * _S:] = out_v.astype(bf16)

    rootv = jnp.broadcast_to(rb[...], (_B, _S))
    ovec[_r(_ML):_ROWS, :] = rootv
    osc[_r(_ML):_ROWS, :] = jnp.zeros((_B, 1), f32)
    opj[_r(_ML):_ROWS, :] = jnp.dot(
        rootv.astype(bf16), wfull_v[:, :2 * _S], preferred_element_type=f32).astype(bf16)

    for L in range(_ML - 1, 0, -1):
        nc = _ML + 1 - L
        n16 = nc * _B
        P = _ML - L
        pml, sbl, psl, ssl = [], [], [], []
        for p in range(P):
            pa = _r(_ML - p)
            sa = _r(_ML - L - p) + L * _B
            nb = (p + 1) * _B
            pml.append(opj[pa:pa + nb, :])
            sbl.append(ipj[sa:sa + nb, 2 * _S:])
            psl.append(osc[pa:pa + nb, :])
            ssl.append(isc[sa:sa + nb, :])
            pa = _r(L + p + 1)
            sa = _r(p + 1)
            na = (_ML - L - p) * _B
            pml.append(opj[pa:pa + na, :])
            sbl.append(ipj[sa:sa + na, 2 * _S:])
            psl.append(osc[pa:pa + na, :])
            ssl.append(isc[sa:sa + na, :])
        out_v = level_update(
            P, n16,
            jnp.concatenate(pml, axis=0), jnp.concatenate(sbl, axis=0),
            jnp.concatenate(psl, axis=0), jnp.concatenate(ssl, axis=0),
            ovec, osc, _r(L))
        row0 = _r(L)
        opj[row0:row0 + n16, :] = jnp.dot(
            out_v.astype(bf16), wfull_v[:, :2 * _S], preferred_element_type=f32).astype(bf16)


def kernel(word_embeddings, W1, b1, W2, b2, bilinear_w, root_bias):
    we = word_embeddings.reshape(_ML * _B, _S)
    w1t = W1.T
    wfull = jnp.concatenate([w1t[:_S, :], bilinear_w[0], w1t[_S:, :]], axis=1)
    f32 = jnp.float32
    ivec, isc, ovec, osc = pl.pallas_call(
        _body,
        out_shape=(
            jax.ShapeDtypeStruct((_ROWS, _S), f32),
            jax.ShapeDtypeStruct((_ROWS, 1), f32),
            jax.ShapeDtypeStruct((_ROWS, _S), f32),
            jax.ShapeDtypeStruct((_ROWS, 1), f32),
        ),
        scratch_shapes=[
            pltpu.VMEM((_ROWS, 4 * _S), jnp.bfloat16),
            pltpu.VMEM((_ROWS, 2 * _S), jnp.bfloat16),
        ],
    )(we, wfull, b1.reshape(1, _S), W2.T, b2.reshape(1, _S),
      root_bias.reshape(1, _S))
    return jnp.concatenate([
        ivec.reshape(_T, _B, _S),
        isc.reshape(_T, _B, 1),
        ovec.reshape(_T, _B, _S),
        osc.reshape(_T, _B, 1),
    ], axis=-1)


# speedup vs baseline: 1.0529x; 1.0529x over previous
import jax
import jax.numpy as jnp
from jax.experimental import pallas as pl
from jax.experimental.pallas import tpu as pltpu

_ML = 12
_B = 16
_S = 768
_T = (_ML + 1) * _ML // 2
_ROWS = _T * _B


def _off(L):
    return _T - (_ML - L + 2) * (_ML - L + 1) // 2


def _r(L):
    return _off(L) * _B


def _body(we, wfull, b1, w2t, b2, rb,
          ivec, isc, ovec, osc,
          ipj, opj):
    f32 = jnp.float32
    bf16 = jnp.bfloat16
    wfull_v = wfull[...]
    b1_v = b1[...]
    w2t_v = w2t[...]
    b2_v = b2[...]

    def level_update(n_slots, n16, pieces, sl, sr, vec_ref, sc_ref, row0):
        h = jnp.concatenate(
            [jax.nn.relu(lp[:, :_S].astype(f32) + rp[:, :_S].astype(f32) + b1_v)
             for lp, rp in pieces], axis=0)
        v = jax.nn.relu(jnp.dot(h, w2t_v, preferred_element_type=f32) + b2_v)
        s = jnp.concatenate(
            [jnp.sum(lp[:, _S:].astype(f32) * rp[:, _S:].astype(f32),
                     axis=1, keepdims=True)
             for lp, rp in pieces], axis=0) + sl + sr
        s3 = s.reshape(n_slots, n16, 1)
        m = jnp.max(s3, axis=0, keepdims=True)
        e = jnp.exp(s3 - m)
        w3 = e / jnp.sum(e, axis=0, keepdims=True)
        out_s = jnp.sum(w3 * s3, axis=0)
        out_v = jnp.sum(w3 * v.reshape(n_slots, n16, _S), axis=0)
        vec_ref[row0:row0 + n16, :] = out_v
        sc_ref[row0:row0 + n16, :] = out_s
        return out_v

    we_v = we[...]
    ivec[0:_ML * _B, :] = we_v
    isc[0:_ML * _B, :] = jnp.zeros((_ML * _B, 1), f32)
    ipj[0:_ML * _B, 0:3 * _S] = jnp.dot(we_v, wfull_v,
                                        preferred_element_type=f32).astype(bf16)
    ipj[0:_ML * _B, 3 * _S:] = we_v.astype(bf16)

    for L in range(2, _ML + 1):
        nc = _ML + 1 - L
        n16 = nc * _B
        c = L - 1
        pieces, sll, srl = [], [], []
        for ci in range(c):
            a = _r(1 + ci)
            b = _r(L - 1 - ci) + (1 + ci) * _B
            pieces.append((ipj[a:a + n16, 0:2 * _S], ipj[b:b + n16, 2 * _S:]))
            sll.append(isc[a:a + n16, :])
            srl.append(isc[b:b + n16, :])
        out_v = level_update(
            c, n16, pieces,
            jnp.concatenate(sll, axis=0), jnp.concatenate(srl, axis=0),
            ivec, isc, _r(L))
        row0 = _r(L)
        ipj[row0:row0 + n16, 0:3 * _S] = jnp.dot(
            out_v, wfull_v, preferred_element_type=f32).astype(bf16)
        ipj[row0:row0 + n16, 3 * _S:] = out_v.astype(bf16)

    rootv = jnp.broadcast_to(rb[...], (_B, _S))
    ovec[_r(_ML):_ROWS, :] = rootv
    osc[_r(_ML):_ROWS, :] = jnp.zeros((_B, 1), f32)
    opj[_r(_ML):_ROWS, :] = jnp.dot(
        rootv, wfull_v[:, :2 * _S], preferred_element_type=f32).astype(bf16)

    for L in range(_ML - 1, 0, -1):
        nc = _ML + 1 - L
        n16 = nc * _B
        P = _ML - L
        pieces, psl, ssl = [], [], []
        for p in range(P):
            pa = _r(_ML - p)
            sa = _r(_ML - L - p) + L * _B
            nb = (p + 1) * _B
            pieces.append((opj[pa:pa + nb, :], ipj[sa:sa + nb, 2 * _S:]))
            psl.append(osc[pa:pa + nb, :])
            ssl.append(isc[sa:sa + nb, :])
            pa = _r(L + p + 1)
            sa = _r(p + 1)
            na = (_ML - L - p) * _B
            pieces.append((opj[pa:pa + na, :], ipj[sa:sa + na, 2 * _S:]))
            psl.append(osc[pa:pa + na, :])
            ssl.append(isc[sa:sa + na, :])
        out_v = level_update(
            P, n16, pieces,
            jnp.concatenate(psl, axis=0), jnp.concatenate(ssl, axis=0),
            ovec, osc, _r(L))
        row0 = _r(L)
        opj[row0:row0 + n16, :] = jnp.dot(
            out_v, wfull_v[:, :2 * _S], preferred_element_type=f32).astype(bf16)


def kernel(word_embeddings, W1, b1, W2, b2, bilinear_w, root_bias):
    we = word_embeddings.reshape(_ML * _B, _S)
    w1t = W1.T
    wfull = jnp.concatenate([w1t[:_S, :], bilinear_w[0], w1t[_S:, :]], axis=1)
    f32 = jnp.float32
    ivec, isc, ovec, osc = pl.pallas_call(
        _body,
        out_shape=(
            jax.ShapeDtypeStruct((_ROWS, _S), f32),
            jax.ShapeDtypeStruct((_ROWS, 1), f32),
            jax.ShapeDtypeStruct((_ROWS, _S), f32),
            jax.ShapeDtypeStruct((_ROWS, 1), f32),
        ),
        scratch_shapes=[
            pltpu.VMEM((_ROWS, 4 * _S), jnp.bfloat16),
            pltpu.VMEM((_ROWS, 2 * _S), jnp.bfloat16),
        ],
    )(we, wfull, b1.reshape(1, _S), W2.T, b2.reshape(1, _S),
      root_bias.reshape(1, _S))
    return jnp.concatenate([
        ivec.reshape(_T, _B, _S),
        isc.reshape(_T, _B, 1),
        ovec.reshape(_T, _B, _S),
        osc.reshape(_T, _B, 1),
    ], axis=-1)


# speedup vs baseline: 1.0777x; 1.0235x over previous
import jax
import jax.numpy as jnp
from jax.experimental import pallas as pl
from jax.experimental.pallas import tpu as pltpu

_ML = 12
_B = 16
_S = 768
_T = (_ML + 1) * _ML // 2
_ROWS = _T * _B


def _off(L):
    return _T - (_ML - L + 2) * (_ML - L + 1) // 2


def _r(L):
    return _off(L) * _B


def _body(we, wfull, b1, w2t, b2, rb,
          ivec, isc, ovec, osc,
          ipj, opj):
    f32 = jnp.float32
    bf16 = jnp.bfloat16
    wfull_v = wfull[...]
    b1_v = b1[...]
    b1cat_v = jnp.concatenate(
        [jnp.zeros((1, 2 * _S), f32), b1_v], axis=1)
    w2t_v = w2t[...].astype(bf16)
    b2_v = b2[...]

    def level_update(n_slots, n16, pieces, sl, sr, vec_ref, sc_ref, row0):
        h = jnp.concatenate(
            [jax.nn.relu(lp[:, :_S] + rp[:, :_S]) for lp, rp in pieces], axis=0)
        v = jax.nn.relu(jnp.dot(h, w2t_v, preferred_element_type=f32) + b2_v)
        s = jnp.concatenate(
            [jnp.sum(lp[:, _S:].astype(f32) * rp[:, _S:].astype(f32),
                     axis=1, keepdims=True)
             for lp, rp in pieces], axis=0) + sl + sr
        s3 = s.reshape(n_slots, n16, 1)
        m = jnp.max(s3, axis=0, keepdims=True)
        e = jnp.exp(s3 - m)
        w3 = e / jnp.sum(e, axis=0, keepdims=True)
        out_s = jnp.sum(w3 * s3, axis=0)
        out_v = jnp.sum(w3 * v.reshape(n_slots, n16, _S), axis=0)
        vec_ref[row0:row0 + n16, :] = out_v
        sc_ref[row0:row0 + n16, :] = out_s
        return out_v

    we_v = we[...]
    ivec[0:_ML * _B, :] = we_v
    isc[0:_ML * _B, :] = jnp.zeros((_ML * _B, 1), f32)
    ipj[0:_ML * _B, 0:3 * _S] = (jnp.dot(we_v, wfull_v,
                                         preferred_element_type=f32)
                                 + b1cat_v).astype(bf16)
    ipj[0:_ML * _B, 3 * _S:] = we_v.astype(bf16)

    for L in range(2, _ML + 1):
        nc = _ML + 1 - L
        n16 = nc * _B
        c = L - 1
        pieces, sll, srl = [], [], []
        for ci in range(c):
            a = _r(1 + ci)
            b = _r(L - 1 - ci) + (1 + ci) * _B
            pieces.append((ipj[a:a + n16, 0:2 * _S], ipj[b:b + n16, 2 * _S:]))
            sll.append(isc[a:a + n16, :])
            srl.append(isc[b:b + n16, :])
        out_v = level_update(
            c, n16, pieces,
            jnp.concatenate(sll, axis=0), jnp.concatenate(srl, axis=0),
            ivec, isc, _r(L))
        row0 = _r(L)
        ipj[row0:row0 + n16, 0:3 * _S] = (jnp.dot(
            out_v, wfull_v, preferred_element_type=f32) + b1cat_v).astype(bf16)
        ipj[row0:row0 + n16, 3 * _S:] = out_v.astype(bf16)

    rootv = jnp.broadcast_to(rb[...], (_B, _S))
    ovec[_r(_ML):_ROWS, :] = rootv
    osc[_r(_ML):_ROWS, :] = jnp.zeros((_B, 1), f32)
    opj[_r(_ML):_ROWS, :] = jnp.dot(
        rootv, wfull_v[:, :2 * _S], preferred_element_type=f32).astype(bf16)

    for L in range(_ML - 1, 0, -1):
        nc = _ML + 1 - L
        n16 = nc * _B
        P = _ML - L
        pieces, psl, ssl = [], [], []
        for p in range(P):
            pa = _r(_ML - p)
            sa = _r(_ML - L - p) + L * _B
            nb = (p + 1) * _B
            pieces.append((opj[pa:pa + nb, :], ipj[sa:sa + nb, 2 * _S:]))
            psl.append(osc[pa:pa + nb, :])
            ssl.append(isc[sa:sa + nb, :])
            pa = _r(L + p + 1)
            sa = _r(p + 1)
            na = (_ML - L - p) * _B
            pieces.append((opj[pa:pa + na, :], ipj[sa:sa + na, 2 * _S:]))
            psl.append(osc[pa:pa + na, :])
            ssl.append(isc[sa:sa + na, :])
        out_v = level_update(
            P, n16, pieces,
            jnp.concatenate(psl, axis=0), jnp.concatenate(ssl, axis=0),
            ovec, osc, _r(L))
        row0 = _r(L)
        opj[row0:row0 + n16, :] = jnp.dot(
            out_v, wfull_v[:, :2 * _S], preferred_element_type=f32).astype(bf16)


def kernel(word_embeddings, W1, b1, W2, b2, bilinear_w, root_bias):
    we = word_embeddings.reshape(_ML * _B, _S)
    w1t = W1.T
    wfull = jnp.concatenate([w1t[:_S, :], bilinear_w[0], w1t[_S:, :]], axis=1)
    f32 = jnp.float32
    ivec, isc, ovec, osc = pl.pallas_call(
        _body,
        out_shape=(
            jax.ShapeDtypeStruct((_ROWS, _S), f32),
            jax.ShapeDtypeStruct((_ROWS, 1), f32),
            jax.ShapeDtypeStruct((_ROWS, _S), f32),
            jax.ShapeDtypeStruct((_ROWS, 1), f32),
        ),
        scratch_shapes=[
            pltpu.VMEM((_ROWS, 4 * _S), jnp.bfloat16),
            pltpu.VMEM((_ROWS, 2 * _S), jnp.bfloat16),
        ],
    )(we, wfull, b1.reshape(1, _S), W2.T, b2.reshape(1, _S),
      root_bias.reshape(1, _S))
    return jnp.concatenate([
        ivec.reshape(_T, _B, _S),
        isc.reshape(_T, _B, 1),
        ovec.reshape(_T, _B, _S),
        osc.reshape(_T, _B, 1),
    ], axis=-1)


# speedup vs baseline: 1.0903x; 1.0117x over previous
import jax
import jax.numpy as jnp
from jax.experimental import pallas as pl
from jax.experimental.pallas import tpu as pltpu

_ML = 12
_B = 16
_S = 768
_T = (_ML + 1) * _ML // 2
_ROWS = _T * _B


def _off(L):
    return _T - (_ML - L + 2) * (_ML - L + 1) // 2


def _r(L):
    return _off(L) * _B


def _body(we, wfull, b1, w2t, b2, rb,
          ivec, isc, ovec, osc,
          ipj, opj):
    f32 = jnp.float32
    bf16 = jnp.bfloat16
    wfull_v = wfull[...]
    b1_v = b1[...]
    b1cat_v = jnp.concatenate(
        [jnp.zeros((1, 2 * _S), f32), b1_v], axis=1)
    w2t_v = w2t[...].astype(bf16)
    b2_v = b2[...]

    def level_update(n_slots, n16, pieces, sl, sr, vec_ref, sc_ref, row0):
        h = jnp.concatenate(
            [jax.nn.relu(lp[:, :_S] + rp[:, :_S]) for lp, rp in pieces], axis=0)
        v = jax.nn.relu(jnp.dot(h, w2t_v, preferred_element_type=f32) + b2_v)
        s = jnp.concatenate(
            [jnp.sum((lp[:, _S:] * rp[:, _S:]).astype(f32),
                     axis=1, keepdims=True)
             for lp, rp in pieces], axis=0) + sl + sr
        s3 = s.reshape(n_slots, n16, 1)
        m = jnp.max(s3, axis=0, keepdims=True)
        e = jnp.exp(s3 - m)
        w3 = e / jnp.sum(e, axis=0, keepdims=True)
        out_s = jnp.sum(w3 * s3, axis=0)
        out_v = jnp.sum(w3 * v.reshape(n_slots, n16, _S), axis=0)
        vec_ref[row0:row0 + n16, :] = out_v
        sc_ref[row0:row0 + n16, :] = out_s
        return out_v

    we_v = we[...]
    ivec[0:_ML * _B, :] = we_v
    isc[0:_ML * _B, :] = jnp.zeros((_ML * _B, 1), f32)
    ipj[0:_ML * _B, 0:3 * _S] = (jnp.dot(we_v, wfull_v,
                                         preferred_element_type=f32)
                                 + b1cat_v).astype(bf16)
    ipj[0:_ML * _B, 3 * _S:] = we_v.astype(bf16)

    for L in range(2, _ML + 1):
        nc = _ML + 1 - L
        n16 = nc * _B
        c = L - 1
        pieces, sll, srl = [], [], []
        for ci in range(c):
            a = _r(1 + ci)
            b = _r(L - 1 - ci) + (1 + ci) * _B
            pieces.append((ipj[a:a + n16, 0:2 * _S], ipj[b:b + n16, 2 * _S:]))
            sll.append(isc[a:a + n16, :])
            srl.append(isc[b:b + n16, :])
        out_v = level_update(
            c, n16, pieces,
            jnp.concatenate(sll, axis=0), jnp.concatenate(srl, axis=0),
            ivec, isc, _r(L))
        row0 = _r(L)
        ipj[row0:row0 + n16, 0:3 * _S] = (jnp.dot(
            out_v, wfull_v, preferred_element_type=f32) + b1cat_v).astype(bf16)
        ipj[row0:row0 + n16, 3 * _S:] = out_v.astype(bf16)

    rootv = jnp.broadcast_to(rb[...], (_B, _S))
    ovec[_r(_ML):_ROWS, :] = rootv
    osc[_r(_ML):_ROWS, :] = jnp.zeros((_B, 1), f32)
    opj[_r(_ML):_ROWS, :] = jnp.dot(
        rootv, wfull_v[:, :2 * _S], preferred_element_type=f32).astype(bf16)

    for L in range(_ML - 1, 0, -1):
        nc = _ML + 1 - L
        n16 = nc * _B
        P = _ML - L
        pieces, psl, ssl = [], [], []
        for p in range(P):
            pa = _r(_ML - p)
            sa = _r(_ML - L - p) + L * _B
            nb = (p + 1) * _B
            pieces.append((opj[pa:pa + nb, :], ipj[sa:sa + nb, 2 * _S:]))
            psl.append(osc[pa:pa + nb, :])
            ssl.append(isc[sa:sa + nb, :])
            pa = _r(L + p + 1)
            sa = _r(p + 1)
            na = (_ML - L - p) * _B
            pieces.append((opj[pa:pa + na, :], ipj[sa:sa + na, 2 * _S:]))
            psl.append(osc[pa:pa + na, :])
            ssl.append(isc[sa:sa + na, :])
        out_v = level_update(
            P, n16, pieces,
            jnp.concatenate(psl, axis=0), jnp.concatenate(ssl, axis=0),
            ovec, osc, _r(L))
        row0 = _r(L)
        opj[row0:row0 + n16, :] = jnp.dot(
            out_v, wfull_v[:, :2 * _S], preferred_element_type=f32).astype(bf16)


def kernel(word_embeddings, W1, b1, W2, b2, bilinear_w, root_bias):
    we = word_embeddings.reshape(_ML * _B, _S)
    w1t = W1.T
    wfull = jnp.concatenate([w1t[:_S, :], bilinear_w[0], w1t[_S:, :]], axis=1)
    f32 = jnp.float32
    ivec, isc, ovec, osc = pl.pallas_call(
        _body,
        out_shape=(
            jax.ShapeDtypeStruct((_ROWS, _S), f32),
            jax.ShapeDtypeStruct((_ROWS, 1), f32),
            jax.ShapeDtypeStruct((_ROWS, _S), f32),
            jax.ShapeDtypeStruct((_ROWS, 1), f32),
        ),
        scratch_shapes=[
            pltpu.VMEM((_ROWS, 4 * _S), jnp.bfloat16),
            pltpu.VMEM((_ROWS, 2 * _S), jnp.bfloat16),
        ],
    )(we, wfull, b1.reshape(1, _S), W2.T, b2.reshape(1, _S),
      root_bias.reshape(1, _S))
    return jnp.concatenate([
        ivec.reshape(_T, _B, _S),
        isc.reshape(_T, _B, 1),
        ovec.reshape(_T, _B, _S),
        osc.reshape(_T, _B, 1),
    ], axis=-1)
